# baseline (speedup 1.0000x reference)
import functools

import jax
import jax.numpy as jnp
from jax import lax
from jax.experimental import pallas as pl
from jax.experimental.pallas import tpu as pltpu
from jax.experimental.pallas import tpu_sc as plsc

_LANES = 16


def _wid(info):
    return lax.axis_index("s") * info.num_cores + lax.axis_index("c")


def _make_sc_init(n_total, d, n_init, info):
    nw = info.num_cores * info.num_subcores
    per = n_init // nw
    mesh = plsc.VectorSubcoreMesh(core_axis_name="c", subcore_axis_name="s")

    @functools.partial(
        pl.kernel,
        out_type=jax.ShapeDtypeStruct((n_total, d), jnp.float32),
        mesh=mesh,
        scratch_types=[
            pltpu.VMEM((per,), jnp.int32),
            pltpu.VMEM((per,), jnp.int32),
            pltpu.VMEM((per, d), jnp.float32),
            pltpu.VMEM((per, d), jnp.float32),
            pltpu.SemaphoreType.DMA,
            pltpu.SemaphoreType.DMA,
        ],
    )
    def init_k(thax_ids_hbm, sine_ids_hbm, thax_tab_hbm, sine_tab_hbm,
               store_hbm, idx_t, idx_s, rows_t, rows_s, sem1, sem2):
        w = _wid(info)
        pltpu.sync_copy(thax_ids_hbm.at[pl.ds(w * per, per)], idx_t)
        pltpu.sync_copy(sine_ids_hbm.at[pl.ds(w * per, per)], idx_s)
        c1 = pltpu.async_copy(thax_tab_hbm.at[idx_t], rows_t, sem1)
        c2 = pltpu.async_copy(sine_tab_hbm.at[idx_s], rows_s, sem2)
        c1.wait()
        c2.wait()
        nvec = d // _LANES

        def body(t, carry):
            i = t // nvec
            k = t % nvec
            sl = pl.ds(k * _LANES, _LANES)
            rows_t[i, sl] = rows_t[i, sl] + rows_s[i, sl]
            return carry

        lax.fori_loop(0, per * nvec, body, 0)
        pltpu.sync_copy(rows_t, store_hbm.at[pl.ds(w * per, per)])

    return init_k


def _make_sc_gather(n_total, d, n_idx, n_layers, layer, info):
    nw = info.num_cores * info.num_subcores
    per = n_idx // nw
    chunk = 128
    nchunks = per // chunk
    half = nw // 2
    mesh = plsc.VectorSubcoreMesh(core_axis_name="c", subcore_axis_name="s")

    @functools.partial(
        pl.kernel,
        out_type=jax.ShapeDtypeStruct((n_idx, d), jnp.float32),
        mesh=mesh,
        scratch_types=[
            pltpu.VMEM((nchunks, chunk), jnp.int32),
            pltpu.VMEM((nchunks, chunk), jnp.int32),
            pltpu.VMEM((per, d), jnp.float32),
            pltpu.SemaphoreType.DMA,
            pltpu.SemaphoreType.DMA,
        ],
    )
    def gather_k(store_hbm, parsflat_hbm, posc_hbm, out_hbm, pidx_v, idx_v,
                 rows_v, semg, semw):
        w = _wid(info)
        pltpu.sync_copy(posc_hbm.at[layer, w], pidx_v)
        fetches = [
            pltpu.async_copy(parsflat_hbm.at[pidx_v.at[j]], idx_v.at[j], semg)
            for j in range(nchunks)
        ]
        for c in fetches:
            c.wait()
        gathers = [
            pltpu.async_copy(store_hbm.at[idx_v.at[j]],
                             rows_v.at[pl.ds(j * chunk, chunk)], semg)
            for j in range(nchunks)
        ]
        writes = []
        for j in range(nchunks):
            gathers[j].wait()
            writes.append(pltpu.async_copy(
                rows_v.at[pl.ds(j * chunk, chunk)],
                out_hbm.at[pl.ds(w * per + j * chunk, chunk)], semw))
        for c in writes:
            c.wait()

    return gather_k


def _softplus_terms(m):
    t = jnp.log1p(jnp.exp(-jnp.abs(m)))
    sp_pos = jnp.maximum(m, 0.0) + t
    sp_neg = jnp.maximum(-m, 0.0) + t
    return sp_pos, sp_neg


def _eval_accumulate(e, ev1, ev2p, evb1, evb2_s, pos2, neg2, acc_ref, d):
    n = e.shape[0]
    hh = jax.lax.dot(e, ev1, preferred_element_type=jnp.float32)
    hh = jnp.maximum(hh + evb1.reshape(1, d), 0.0)
    m = jax.lax.dot(hh, ev2p, preferred_element_type=jnp.float32) + evb2_s
    maskf = (lax.broadcasted_iota(jnp.int32, (n, 128), 1) == 0).astype(
        jnp.float32)
    pos_b = pos2 * maskf
    neg_b = neg2 * maskf
    sp_pos, sp_neg = _softplus_terms(m)
    is_pos = (m >= 0.0).astype(jnp.float32)
    acc_ref[0, :] += jnp.sum(pos_b * sp_neg, axis=0)
    acc_ref[1, :] += jnp.sum(neg_b * sp_pos, axis=0)
    acc_ref[2, :] += jnp.sum(pos_b, axis=0)
    acc_ref[3, :] += jnp.sum(neg_b, axis=0)
    acc_ref[4, :] += jnp.sum(pos_b * is_pos, axis=0)
    acc_ref[5, :] += jnp.sum(neg_b * (1.0 - is_pos), axis=0)


def _make_tc_mlp(n_total, d, npl, r_rules, base_row):
    npr = npl // r_rules
    base_block = base_row // npr
    nb = base_row // npr

    def body(pa_ref, pb_ref, w1_ref, b1_ref, w2_ref, b2_ref,
             ev1_ref, ev2p_ref, evb1_ref, evb2_ref, pos_ref, neg_ref,
             store_ref, acc_in_ref, out_ref, acc_out_ref, accv_ref):
        r = pl.program_id(0)

        @pl.when(r == 0)
        def _():
            accv_ref[...] = jnp.zeros((8, 128), jnp.float32)

        w1 = w1_ref[0]
        h = jax.lax.dot(pa_ref[...], w1[:d], preferred_element_type=jnp.float32)
        h = h + jax.lax.dot(pb_ref[...], w1[d:],
                            preferred_element_type=jnp.float32)
        h = jnp.maximum(h + b1_ref[0], 0.0)
        e = jax.lax.dot(h, w2_ref[0], preferred_element_type=jnp.float32)
        e = jnp.maximum(e + b2_ref[0], 0.0)
        out_ref[...] = e
        _eval_accumulate(e, ev1_ref[...], ev2p_ref[...], evb1_ref[...],
                         evb2_ref[0], pos_ref[...], neg_ref[...], accv_ref, d)

        @pl.when(r == r_rules - 1)
        def _():
            acc_out_ref[...] = acc_in_ref[...] + accv_ref[...]

    return pl.pallas_call(
        body,
        grid=(r_rules,),
        in_specs=[
            pl.BlockSpec((npr, d), lambda r: (r, 0)),
            pl.BlockSpec((npr, d), lambda r: (r_rules + r, 0)),
            pl.BlockSpec((1, 2 * d, d), lambda r: (r, 0, 0)),
            pl.BlockSpec((1, 1, d), lambda r: (r, 0, 0)),
            pl.BlockSpec((1, d, d), lambda r: (r, 0, 0)),
            pl.BlockSpec((1, 1, d), lambda r: (r, 0, 0)),
            pl.BlockSpec((d, d), lambda r: (0, 0)),
            pl.BlockSpec((d, 128), lambda r: (0, 0)),
            pl.BlockSpec((d,), lambda r: (0,)),
            pl.BlockSpec(memory_space=pltpu.MemorySpace.SMEM),
            pl.BlockSpec((npr, 1), lambda r: (nb + r, 0)),
            pl.BlockSpec((npr, 1), lambda r: (nb + r, 0)),
            pl.BlockSpec(memory_space=pltpu.MemorySpace.HBM),
            pl.BlockSpec((8, 128), lambda r: (0, 0)),
        ],
        out_specs=[
            pl.BlockSpec((npr, d), lambda r: (base_block + r, 0)),
            pl.BlockSpec((8, 128), lambda r: (0, 0)),
        ],
        out_shape=[
            jax.ShapeDtypeStruct((n_total, d), jnp.float32),
            jax.ShapeDtypeStruct((8, 128), jnp.float32),
        ],
        scratch_shapes=[pltpu.VMEM((8, 128), jnp.float32)],
        input_output_aliases={12: 0, 13: 1},
    )


def _make_tc_final(n_total, d, n_init, blk):
    nblocks = n_init // blk

    def body(store_ref, ev1_ref, ev2p_ref, evb1_ref, evb2_ref,
             pos_ref, neg_ref, acc_in_ref,
             loss_ref, pok_ref, nok_ref, accv_ref):
        i = pl.program_id(0)

        @pl.when(i == 0)
        def _():
            accv_ref[...] = jnp.zeros((8, 128), jnp.float32)

        _eval_accumulate(store_ref[...], ev1_ref[...], ev2p_ref[...],
                         evb1_ref[...], evb2_ref[0], pos_ref[...],
                         neg_ref[...], accv_ref, d)

        @pl.when(i == nblocks - 1)
        def _():
            s = acc_in_ref[...] + accv_ref[...]
            a = jnp.sum(s[0, :])
            b = jnp.sum(s[1, :])
            tot_pos = jnp.sum(s[2, :])
            tot_neg = jnp.sum(s[3, :])
            loss_ref[...] = ((tot_neg / tot_pos) * a + b).reshape(1, 1)
            pok_ref[...] = jnp.sum(s[4, :]).reshape(1, 1)
            nok_ref[...] = jnp.sum(s[5, :]).reshape(1, 1)

    return pl.pallas_call(
        body,
        grid=(nblocks,),
        in_specs=[
            pl.BlockSpec((blk, d), lambda i: (i, 0)),
            pl.BlockSpec((d, d), lambda i: (0, 0)),
            pl.BlockSpec((d, 128), lambda i: (0, 0)),
            pl.BlockSpec((d,), lambda i: (0,)),
            pl.BlockSpec(memory_space=pltpu.MemorySpace.SMEM),
            pl.BlockSpec((blk, 1), lambda i: (i, 0)),
            pl.BlockSpec((blk, 1), lambda i: (i, 0)),
            pl.BlockSpec((8, 128), lambda i: (0, 0)),
        ],
        out_specs=[
            pl.BlockSpec((1, 1), lambda i: (0, 0)),
            pl.BlockSpec((1, 1), lambda i: (0, 0)),
            pl.BlockSpec((1, 1), lambda i: (0, 0)),
        ],
        out_shape=[
            jax.ShapeDtypeStruct((1, 1), jnp.float32),
            jax.ShapeDtypeStruct((1, 1), jnp.float32),
            jax.ShapeDtypeStruct((1, 1), jnp.float32),
        ],
        scratch_shapes=[pltpu.VMEM((8, 128), jnp.float32)],
    )


def kernel(thax_ids, sine_ids, pars, pos_vals, neg_vals, thax_table,
           sine_table, W1, b1, W2, b2, Ev1, evb1, Ev2, evb2):
    n_init = thax_ids.shape[0]
    n_layers, npl = pars.shape[0], pars.shape[1]
    d = thax_table.shape[1]
    r_rules = W1.shape[0]
    n_total = pos_vals.shape[0]
    info = plsc.get_sparse_core_info()
    nw = info.num_cores * info.num_subcores

    init_k = _make_sc_init(n_total, d, n_init, info)
    store = init_k(thax_ids.astype(jnp.int32), sine_ids.astype(jnp.int32),
                   thax_table, sine_table)

    pars_flat = pars.astype(jnp.int32).reshape(-1)
    half = nw // 2
    per = 2 * npl // nw
    w_ids = jnp.arange(nw, dtype=jnp.int32)
    col = w_ids // half
    wa = w_ids % half
    k_ids = jnp.arange(per, dtype=jnp.int32).reshape(per // 128, 128)
    l_ids = jnp.arange(n_layers, dtype=jnp.int32)
    posc = (l_ids[:, None, None, None] * (2 * npl)
            + 2 * (wa[None, :, None, None] * per + k_ids[None, None])
            + col[None, :, None, None])
    b1r = b1.reshape(r_rules, 1, d)
    b2r = b2.reshape(r_rules, 1, d)
    ev2p = jnp.pad(Ev2, ((0, 0), (0, 127)))
    pos2 = pos_vals.reshape(-1, 1)
    neg2 = neg_vals.reshape(-1, 1)
    acc = jnp.zeros((8, 128), jnp.float32)
    for l in range(n_layers):
        gather_k = _make_sc_gather(n_total, d, 2 * npl, n_layers, l, info)
        p = gather_k(store, pars_flat, posc)
        mlp_k = _make_tc_mlp(n_total, d, npl, r_rules, n_init + l * npl)
        store, acc = mlp_k(p, p, W1, b1r, W2, b2r, Ev1, ev2p, evb1, evb2,
                           pos2, neg2, store, acc)

    final_k = _make_tc_final(n_total, d, n_init, 512)
    loss2, pok2, nok2 = final_k(store, Ev1, ev2p, evb1, evb2,
                                pos2, neg2, acc)
    return loss2.reshape(1), pok2[0, 0], nok2[0, 0]


# speedup vs baseline: 1.2087x; 1.2087x over previous
import functools

import jax
import jax.numpy as jnp
from jax import lax
from jax.experimental import pallas as pl
from jax.experimental.pallas import tpu as pltpu
from jax.experimental.pallas import tpu_sc as plsc

_LANES = 16


def _wid(info):
    return lax.axis_index("s") * info.num_cores + lax.axis_index("c")


def _make_sc_init(n_total, d, n_init, info):
    nw = info.num_cores * info.num_subcores
    per = n_init // nw
    mesh = plsc.VectorSubcoreMesh(core_axis_name="c", subcore_axis_name="s")

    @functools.partial(
        pl.kernel,
        out_type=jax.ShapeDtypeStruct((n_total, d), jnp.float32),
        mesh=mesh,
        scratch_types=[
            pltpu.VMEM((per,), jnp.int32),
            pltpu.VMEM((per,), jnp.int32),
            pltpu.VMEM((per, d), jnp.float32),
            pltpu.VMEM((per, d), jnp.float32),
            pltpu.SemaphoreType.DMA,
            pltpu.SemaphoreType.DMA,
        ],
    )
    def init_k(thax_ids_hbm, sine_ids_hbm, thax_tab_hbm, sine_tab_hbm,
               store_hbm, idx_t, idx_s, rows_t, rows_s, sem1, sem2):
        w = _wid(info)
        pltpu.sync_copy(thax_ids_hbm.at[pl.ds(w * per, per)], idx_t)
        pltpu.sync_copy(sine_ids_hbm.at[pl.ds(w * per, per)], idx_s)
        c1 = pltpu.async_copy(thax_tab_hbm.at[idx_t], rows_t, sem1)
        c2 = pltpu.async_copy(sine_tab_hbm.at[idx_s], rows_s, sem2)
        c1.wait()
        c2.wait()
        nvec = d // _LANES

        def body(t, carry):
            i = t // nvec
            k = t % nvec
            sl = pl.ds(k * _LANES, _LANES)
            rows_t[i, sl] = rows_t[i, sl] + rows_s[i, sl]
            return carry

        lax.fori_loop(0, per * nvec, body, 0)
        pltpu.sync_copy(rows_t, store_hbm.at[pl.ds(w * per, per)])

    return init_k


def _make_sc_gather(n_total, d, n_idx, n_layers, layer, info):
    nw = info.num_cores * info.num_subcores
    per = n_idx // nw
    chunk = 128
    nchunks = per // chunk
    half = nw // 2
    mesh = plsc.VectorSubcoreMesh(core_axis_name="c", subcore_axis_name="s")

    @functools.partial(
        pl.kernel,
        out_type=jax.ShapeDtypeStruct((n_idx, d), jnp.float32),
        mesh=mesh,
        scratch_types=[
            pltpu.VMEM((nchunks, chunk), jnp.int32),
            pltpu.VMEM((per, d), jnp.float32),
            pltpu.SemaphoreType.DMA,
            pltpu.SemaphoreType.DMA,
        ],
    )
    def gather_k(store_hbm, idx_hbm, out_hbm, idx_v, rows_v, semg, semw):
        w = _wid(info)
        pltpu.sync_copy(idx_hbm.at[layer, w], idx_v)
        gathers = [
            pltpu.async_copy(store_hbm.at[idx_v.at[j]],
                             rows_v.at[pl.ds(j * chunk, chunk)], semg)
            for j in range(nchunks)
        ]
        writes = []
        for j in range(nchunks):
            gathers[j].wait()
            writes.append(pltpu.async_copy(
                rows_v.at[pl.ds(j * chunk, chunk)],
                out_hbm.at[pl.ds(w * per + j * chunk, chunk)], semw))
        for c in writes:
            c.wait()

    return gather_k


def _softplus_terms(m):
    t = jnp.log1p(jnp.exp(-jnp.abs(m)))
    sp_pos = jnp.maximum(m, 0.0) + t
    sp_neg = jnp.maximum(-m, 0.0) + t
    return sp_pos, sp_neg


def _eval_accumulate(e, ev1, ev2p, evb1, evb2_s, pos_row, neg_row, acc_ref, d):
    n = e.shape[0]
    hh = jax.lax.dot(e, ev1, preferred_element_type=jnp.float32)
    hh = jnp.maximum(hh + evb1.reshape(1, d), 0.0)
    m = jax.lax.dot(hh, ev2p, preferred_element_type=jnp.float32) + evb2_s
    sp_pos, sp_neg = _softplus_terms(m)
    is_pos = (m >= 0.0).astype(jnp.float32)
    pn = jnp.concatenate([pos_row.reshape(1, n), neg_row.reshape(1, n)],
                         axis=0)
    dot = functools.partial(jax.lax.dot, preferred_element_type=jnp.float32)
    acc_ref[0:2, :] += dot(pn, sp_neg)
    acc_ref[2:4, :] += dot(pn, sp_pos)
    acc_ref[4:6, :] += dot(pn, jnp.ones_like(m))
    acc_ref[6:8, :] += dot(pn, is_pos)


def _make_tc_mlp(n_total, d, npl, r_rules, base_row):
    npr = npl // r_rules
    base_block = base_row // npr
    nb = base_row // npr

    def body(pa_ref, pb_ref, w1_ref, b1_ref, w2_ref, b2_ref,
             ev1_ref, ev2p_ref, evb1_ref, evb2_ref, pos_ref, neg_ref,
             store_ref, acc_in_ref, out_ref, acc_out_ref, accv_ref):
        r = pl.program_id(0)

        @pl.when(r == 0)
        def _():
            accv_ref[...] = jnp.zeros((8, 128), jnp.float32)

        w1 = w1_ref[0]
        h = jax.lax.dot(pa_ref[...], w1[:d], preferred_element_type=jnp.float32)
        h = h + jax.lax.dot(pb_ref[...], w1[d:],
                            preferred_element_type=jnp.float32)
        h = jnp.maximum(h + b1_ref[0], 0.0)
        e = jax.lax.dot(h, w2_ref[0], preferred_element_type=jnp.float32)
        e = jnp.maximum(e + b2_ref[0], 0.0)
        out_ref[...] = e
        _eval_accumulate(e, ev1_ref[...], ev2p_ref[...], evb1_ref[...],
                         evb2_ref[0], pos_ref[...], neg_ref[...], accv_ref, d)

        @pl.when(r == r_rules - 1)
        def _():
            acc_out_ref[...] = acc_in_ref[...] + accv_ref[...]

    return pl.pallas_call(
        body,
        grid=(r_rules,),
        in_specs=[
            pl.BlockSpec((npr, d), lambda r: (r, 0)),
            pl.BlockSpec((npr, d), lambda r: (r_rules + r, 0)),
            pl.BlockSpec((1, 2 * d, d), lambda r: (r, 0, 0)),
            pl.BlockSpec((1, 1, d), lambda r: (r, 0, 0)),
            pl.BlockSpec((1, d, d), lambda r: (r, 0, 0)),
            pl.BlockSpec((1, 1, d), lambda r: (r, 0, 0)),
            pl.BlockSpec((d, d), lambda r: (0, 0)),
            pl.BlockSpec((d, 128), lambda r: (0, 0)),
            pl.BlockSpec((d,), lambda r: (0,)),
            pl.BlockSpec(memory_space=pltpu.MemorySpace.SMEM),
            pl.BlockSpec((1, 1, npr), lambda r: (nb + r, 0, 0)),
            pl.BlockSpec((1, 1, npr), lambda r: (nb + r, 0, 0)),
            pl.BlockSpec(memory_space=pltpu.MemorySpace.HBM),
            pl.BlockSpec((8, 128), lambda r: (0, 0)),
        ],
        out_specs=[
            pl.BlockSpec((npr, d), lambda r: (base_block + r, 0)),
            pl.BlockSpec((8, 128), lambda r: (0, 0)),
        ],
        out_shape=[
            jax.ShapeDtypeStruct((n_total, d), jnp.float32),
            jax.ShapeDtypeStruct((8, 128), jnp.float32),
        ],
        scratch_shapes=[pltpu.VMEM((8, 128), jnp.float32)],
        input_output_aliases={12: 0, 13: 1},
    )


def _make_tc_final(n_total, d, n_init, blk):
    nblocks = n_init // blk

    def body(store_ref, ev1_ref, ev2p_ref, evb1_ref, evb2_ref,
             pos_ref, neg_ref, acc_in_ref,
             loss_ref, pok_ref, nok_ref, accv_ref):
        i = pl.program_id(0)

        @pl.when(i == 0)
        def _():
            accv_ref[...] = jnp.zeros((8, 128), jnp.float32)

        _eval_accumulate(store_ref[...], ev1_ref[...], ev2p_ref[...],
                         evb1_ref[...], evb2_ref[0], pos_ref[...],
                         neg_ref[...], accv_ref, d)

        @pl.when(i == nblocks - 1)
        def _():
            lane0 = (lax.broadcasted_iota(jnp.int32, (8, 128), 1) == 0)
            s = jnp.where(lane0, acc_in_ref[...] + accv_ref[...], 0.0)
            a = jnp.sum(s[0, :])
            b = jnp.sum(s[3, :])
            tot_pos = jnp.sum(s[4, :])
            tot_neg = jnp.sum(s[5, :])
            loss_ref[...] = ((tot_neg / tot_pos) * a + b).reshape(1, 1)
            pok_ref[...] = jnp.sum(s[6, :]).reshape(1, 1)
            nok_ref[...] = (tot_neg - jnp.sum(s[7, :])).reshape(1, 1)

    return pl.pallas_call(
        body,
        grid=(nblocks,),
        in_specs=[
            pl.BlockSpec((blk, d), lambda i: (i, 0)),
            pl.BlockSpec((d, d), lambda i: (0, 0)),
            pl.BlockSpec((d, 128), lambda i: (0, 0)),
            pl.BlockSpec((d,), lambda i: (0,)),
            pl.BlockSpec(memory_space=pltpu.MemorySpace.SMEM),
            pl.BlockSpec((1, 1, blk), lambda i: (i, 0, 0)),
            pl.BlockSpec((1, 1, blk), lambda i: (i, 0, 0)),
            pl.BlockSpec((8, 128), lambda i: (0, 0)),
        ],
        out_specs=[
            pl.BlockSpec((1, 1), lambda i: (0, 0)),
            pl.BlockSpec((1, 1), lambda i: (0, 0)),
            pl.BlockSpec((1, 1), lambda i: (0, 0)),
        ],
        out_shape=[
            jax.ShapeDtypeStruct((1, 1), jnp.float32),
            jax.ShapeDtypeStruct((1, 1), jnp.float32),
            jax.ShapeDtypeStruct((1, 1), jnp.float32),
        ],
        scratch_shapes=[pltpu.VMEM((8, 128), jnp.float32)],
    )


def kernel(thax_ids, sine_ids, pars, pos_vals, neg_vals, thax_table,
           sine_table, W1, b1, W2, b2, Ev1, evb1, Ev2, evb2):
    n_init = thax_ids.shape[0]
    n_layers, npl = pars.shape[0], pars.shape[1]
    d = thax_table.shape[1]
    r_rules = W1.shape[0]
    n_total = pos_vals.shape[0]
    info = plsc.get_sparse_core_info()
    nw = info.num_cores * info.num_subcores

    init_k = _make_sc_init(n_total, d, n_init, info)
    store = init_k(thax_ids.astype(jnp.int32), sine_ids.astype(jnp.int32),
                   thax_table, sine_table)

    idx_all = pars.astype(jnp.int32).transpose(0, 2, 1).reshape(
        n_layers, nw, -1, 128)
    b1r = b1.reshape(r_rules, 1, d)
    b2r = b2.reshape(r_rules, 1, d)
    ev2p = jnp.pad(Ev2, ((0, 0), (0, 127)))
    pos3 = pos_vals.reshape(-1, 1, 512)
    neg3 = neg_vals.reshape(-1, 1, 512)
    acc = jnp.zeros((8, 128), jnp.float32)
    for l in range(n_layers):
        gather_k = _make_sc_gather(n_total, d, 2 * npl, n_layers, l, info)
        p = gather_k(store, idx_all)
        mlp_k = _make_tc_mlp(n_total, d, npl, r_rules, n_init + l * npl)
        store, acc = mlp_k(p, p, W1, b1r, W2, b2r, Ev1, ev2p, evb1, evb2,
                           pos3, neg3, store, acc)

    final_k = _make_tc_final(n_total, d, n_init, 512)
    loss2, pok2, nok2 = final_k(store, Ev1, ev2p, evb1, evb2,
                                pos3, neg3, acc)
    return loss2.reshape(1), pok2[0, 0], nok2[0, 0]


# speedup vs baseline: 1.2386x; 1.0248x over previous
import functools

import jax
import jax.numpy as jnp
from jax import lax
from jax.experimental import pallas as pl
from jax.experimental.pallas import tpu as pltpu
from jax.experimental.pallas import tpu_sc as plsc

_LANES = 16


def _wid(info):
    return lax.axis_index("s") * info.num_cores + lax.axis_index("c")


def _make_sc_init(n_total, d, n_init, info):
    nw = info.num_cores * info.num_subcores
    per = n_init // nw
    mesh = plsc.VectorSubcoreMesh(core_axis_name="c", subcore_axis_name="s")

    @functools.partial(
        pl.kernel,
        out_type=jax.ShapeDtypeStruct((n_total, d), jnp.float32),
        mesh=mesh,
        scratch_types=[
            pltpu.VMEM((per,), jnp.int32),
            pltpu.VMEM((per,), jnp.int32),
            pltpu.VMEM((per, d), jnp.float32),
            pltpu.VMEM((per, d), jnp.float32),
            pltpu.SemaphoreType.DMA,
            pltpu.SemaphoreType.DMA,
        ],
    )
    def init_k(thax_ids_hbm, sine_ids_hbm, thax_tab_hbm, sine_tab_hbm,
               store_hbm, idx_t, idx_s, rows_t, rows_s, sem1, sem2):
        w = _wid(info)
        pltpu.sync_copy(thax_ids_hbm.at[pl.ds(w * per, per)], idx_t)
        pltpu.sync_copy(sine_ids_hbm.at[pl.ds(w * per, per)], idx_s)
        c1 = pltpu.async_copy(thax_tab_hbm.at[idx_t], rows_t, sem1)
        c2 = pltpu.async_copy(sine_tab_hbm.at[idx_s], rows_s, sem2)
        c1.wait()
        c2.wait()
        nvec = d // _LANES

        def body(t, carry):
            i = t // nvec
            k = t % nvec
            sl = pl.ds(k * _LANES, _LANES)
            rows_t[i, sl] = rows_t[i, sl] + rows_s[i, sl]
            return carry

        lax.fori_loop(0, per * nvec, body, 0)
        pltpu.sync_copy(rows_t, store_hbm.at[pl.ds(w * per, per)])

    return init_k


def _make_sc_gather(n_total, d, n_idx, n_layers, layer, info):
    nw = info.num_cores * info.num_subcores
    per = n_idx // nw
    chunk = 128
    nchunks = per // chunk
    half = nw // 2
    mesh = plsc.VectorSubcoreMesh(core_axis_name="c", subcore_axis_name="s")

    @functools.partial(
        pl.kernel,
        out_type=jax.ShapeDtypeStruct((n_idx, d), jnp.float32),
        mesh=mesh,
        scratch_types=[
            pltpu.VMEM((nchunks, chunk), jnp.int32),
            pltpu.VMEM((per, d), jnp.float32),
            pltpu.SemaphoreType.DMA,
            pltpu.SemaphoreType.DMA,
        ],
    )
    def gather_k(store_hbm, idx_hbm, out_hbm, idx_v, rows_v, semg, semw):
        w = _wid(info)
        pltpu.sync_copy(idx_hbm.at[layer, w], idx_v)
        gathers = [
            pltpu.async_copy(store_hbm.at[idx_v.at[j]],
                             rows_v.at[pl.ds(j * chunk, chunk)], semg)
            for j in range(nchunks)
        ]
        writes = []
        for j in range(nchunks):
            gathers[j].wait()
            writes.append(pltpu.async_copy(
                rows_v.at[pl.ds(j * chunk, chunk)],
                out_hbm.at[pl.ds(w * per + j * chunk, chunk)], semw))
        for c in writes:
            c.wait()

    return gather_k


def _softplus_terms(m):
    t = jnp.log1p(jnp.exp(-jnp.abs(m)))
    sp_pos = jnp.maximum(m, 0.0) + t
    sp_neg = jnp.maximum(-m, 0.0) + t
    return sp_pos, sp_neg


def _eval_accumulate(e, ev1, ev2p, evb1, evb2_s, pos_row, neg_row, acc_ref, d):
    n = e.shape[0]
    hh = jax.lax.dot(e, ev1, preferred_element_type=jnp.float32)
    hh = jnp.maximum(hh + evb1.reshape(1, d), 0.0)
    m = jax.lax.dot(hh, ev2p, preferred_element_type=jnp.float32) + evb2_s
    sp_pos, sp_neg = _softplus_terms(m)
    is_pos = (m >= 0.0).astype(jnp.float32)
    pn = jnp.concatenate([pos_row.reshape(1, n), neg_row.reshape(1, n)],
                         axis=0)
    dot = functools.partial(jax.lax.dot, preferred_element_type=jnp.float32)
    acc_ref[0:2, :] += dot(pn, sp_neg)
    acc_ref[2:4, :] += dot(pn, sp_pos)
    acc_ref[4:6, :] += dot(pn, jnp.ones_like(m))
    acc_ref[6:8, :] += dot(pn, is_pos)


def _make_tc_mlp(n_total, d, npl, r_rules, base_row):
    npr = npl // r_rules
    base_block = base_row // npr
    nb = base_row // npr

    def body(pa_ref, pb_ref, w1_ref, b1_ref, w2_ref, b2_ref,
             ev1_ref, ev2p_ref, evb1_ref, evb2_ref, pos_ref, neg_ref,
             store_ref, acc_in_ref, out_ref, acc_out_ref, accv_ref):
        r = pl.program_id(0)

        @pl.when(r == 0)
        def _():
            accv_ref[...] = jnp.zeros((8, 128), jnp.float32)

        w1 = w1_ref[0]
        pa = pa_ref[...].astype(jnp.bfloat16)
        pb = pb_ref[...].astype(jnp.bfloat16)
        h = jax.lax.dot(pa, w1[:d], preferred_element_type=jnp.float32)
        h = h + jax.lax.dot(pb, w1[d:], preferred_element_type=jnp.float32)
        h = jnp.maximum(h + b1_ref[0], 0.0)
        e = jax.lax.dot(h.astype(jnp.bfloat16), w2_ref[0],
                        preferred_element_type=jnp.float32)
        e = jnp.maximum(e + b2_ref[0], 0.0)
        out_ref[...] = e
        _eval_accumulate(e, ev1_ref[...], ev2p_ref[...], evb1_ref[...],
                         evb2_ref[0], pos_ref[...], neg_ref[...], accv_ref, d)

        @pl.when(r == r_rules - 1)
        def _():
            acc_out_ref[...] = acc_in_ref[...] + accv_ref[...]

    return pl.pallas_call(
        body,
        grid=(r_rules,),
        in_specs=[
            pl.BlockSpec((npr, d), lambda r: (r, 0)),
            pl.BlockSpec((npr, d), lambda r: (r_rules + r, 0)),
            pl.BlockSpec((1, 2 * d, d), lambda r: (r, 0, 0)),
            pl.BlockSpec((1, 1, d), lambda r: (r, 0, 0)),
            pl.BlockSpec((1, d, d), lambda r: (r, 0, 0)),
            pl.BlockSpec((1, 1, d), lambda r: (r, 0, 0)),
            pl.BlockSpec((d, d), lambda r: (0, 0)),
            pl.BlockSpec((d, 128), lambda r: (0, 0)),
            pl.BlockSpec((d,), lambda r: (0,)),
            pl.BlockSpec(memory_space=pltpu.MemorySpace.SMEM),
            pl.BlockSpec((1, 1, npr), lambda r: (nb + r, 0, 0)),
            pl.BlockSpec((1, 1, npr), lambda r: (nb + r, 0, 0)),
            pl.BlockSpec(memory_space=pltpu.MemorySpace.HBM),
            pl.BlockSpec((8, 128), lambda r: (0, 0)),
        ],
        out_specs=[
            pl.BlockSpec((npr, d), lambda r: (base_block + r, 0)),
            pl.BlockSpec((8, 128), lambda r: (0, 0)),
        ],
        out_shape=[
            jax.ShapeDtypeStruct((n_total, d), jnp.float32),
            jax.ShapeDtypeStruct((8, 128), jnp.float32),
        ],
        scratch_shapes=[pltpu.VMEM((8, 128), jnp.float32)],
        input_output_aliases={12: 0, 13: 1},
    )


def _make_tc_final(n_total, d, n_init, blk):
    nblocks = n_init // blk

    def body(store_ref, ev1_ref, ev2p_ref, evb1_ref, evb2_ref,
             pos_ref, neg_ref, acc_in_ref,
             loss_ref, pok_ref, nok_ref, accv_ref):
        i = pl.program_id(0)

        @pl.when(i == 0)
        def _():
            accv_ref[...] = jnp.zeros((8, 128), jnp.float32)

        _eval_accumulate(store_ref[...], ev1_ref[...], ev2p_ref[...],
                         evb1_ref[...], evb2_ref[0], pos_ref[...],
                         neg_ref[...], accv_ref, d)

        @pl.when(i == nblocks - 1)
        def _():
            lane0 = (lax.broadcasted_iota(jnp.int32, (8, 128), 1) == 0)
            s = jnp.where(lane0, acc_in_ref[...] + accv_ref[...], 0.0)
            a = jnp.sum(s[0, :])
            b = jnp.sum(s[3, :])
            tot_pos = jnp.sum(s[4, :])
            tot_neg = jnp.sum(s[5, :])
            loss_ref[...] = ((tot_neg / tot_pos) * a + b).reshape(1, 1)
            pok_ref[...] = jnp.sum(s[6, :]).reshape(1, 1)
            nok_ref[...] = (tot_neg - jnp.sum(s[7, :])).reshape(1, 1)

    return pl.pallas_call(
        body,
        grid=(nblocks,),
        in_specs=[
            pl.BlockSpec((blk, d), lambda i: (i, 0)),
            pl.BlockSpec((d, d), lambda i: (0, 0)),
            pl.BlockSpec((d, 128), lambda i: (0, 0)),
            pl.BlockSpec((d,), lambda i: (0,)),
            pl.BlockSpec(memory_space=pltpu.MemorySpace.SMEM),
            pl.BlockSpec((1, 1, blk), lambda i: (i, 0, 0)),
            pl.BlockSpec((1, 1, blk), lambda i: (i, 0, 0)),
            pl.BlockSpec((8, 128), lambda i: (0, 0)),
        ],
        out_specs=[
            pl.BlockSpec((1, 1), lambda i: (0, 0)),
            pl.BlockSpec((1, 1), lambda i: (0, 0)),
            pl.BlockSpec((1, 1), lambda i: (0, 0)),
        ],
        out_shape=[
            jax.ShapeDtypeStruct((1, 1), jnp.float32),
            jax.ShapeDtypeStruct((1, 1), jnp.float32),
            jax.ShapeDtypeStruct((1, 1), jnp.float32),
        ],
        scratch_shapes=[pltpu.VMEM((8, 128), jnp.float32)],
    )


def kernel(thax_ids, sine_ids, pars, pos_vals, neg_vals, thax_table,
           sine_table, W1, b1, W2, b2, Ev1, evb1, Ev2, evb2):
    n_init = thax_ids.shape[0]
    n_layers, npl = pars.shape[0], pars.shape[1]
    d = thax_table.shape[1]
    r_rules = W1.shape[0]
    n_total = pos_vals.shape[0]
    info = plsc.get_sparse_core_info()
    nw = info.num_cores * info.num_subcores

    init_k = _make_sc_init(n_total, d, n_init, info)
    store = init_k(thax_ids.astype(jnp.int32), sine_ids.astype(jnp.int32),
                   thax_table, sine_table)

    idx_all = pars.astype(jnp.int32).transpose(0, 2, 1).reshape(
        n_layers, nw, -1, 128)
    w1b = W1.astype(jnp.bfloat16)
    w2b = W2.astype(jnp.bfloat16)
    b1r = b1.reshape(r_rules, 1, d)
    b2r = b2.reshape(r_rules, 1, d)
    ev2p = jnp.pad(Ev2, ((0, 0), (0, 127)))
    pos3 = pos_vals.reshape(-1, 1, 512)
    neg3 = neg_vals.reshape(-1, 1, 512)
    acc = jnp.zeros((8, 128), jnp.float32)
    for l in range(n_layers):
        gather_k = _make_sc_gather(n_total, d, 2 * npl, n_layers, l, info)
        p = gather_k(store, idx_all)
        mlp_k = _make_tc_mlp(n_total, d, npl, r_rules, n_init + l * npl)
        store, acc = mlp_k(p, p, w1b, b1r, w2b, b2r, Ev1, ev2p, evb1, evb2,
                           pos3, neg3, store, acc)

    final_k = _make_tc_final(n_total, d, n_init, 512)
    loss2, pok2, nok2 = final_k(store, Ev1, ev2p, evb1, evb2,
                                pos3, neg3, acc)
    return loss2.reshape(1), pok2[0, 0], nok2[0, 0]
